# baseline (speedup 1.0000x reference)
import functools

import jax
import jax.numpy as jnp
from jax import lax
from jax.experimental import pallas as pl
from jax.experimental.pallas import tpu as pltpu
from jax.experimental.pallas import tpu_sc as plsc

_NC = 2
_NS = 16
_NW = _NC * _NS

_B = 4096 * 50
_D = 32
_CHUNK = 128
_PER_W = _B // _NW
_NCHUNK = _PER_W // _CHUNK


def _body(table_hbm, idx_hbm, out_hbm, idx_v, rows_v, sem):
    wid = lax.axis_index("s") * _NC + lax.axis_index("c")
    base = wid * _PER_W
    pltpu.sync_copy(idx_hbm.at[wid], idx_v)

    def step(j, carry):
        pltpu.async_copy(table_hbm.at[idx_v.at[j]], rows_v, sem).wait()
        pltpu.sync_copy(rows_v, out_hbm.at[pl.ds(base + j * _CHUNK, _CHUNK)])
        return carry

    lax.fori_loop(0, _NCHUNK, step, 0)


@jax.jit
def kernel(embeddings, inputs):
    idx3 = inputs.reshape(_NW, _NCHUNK, _CHUNK)
    run = pl.kernel(
        _body,
        out_type=jax.ShapeDtypeStruct((_B, _D), jnp.float32),
        mesh=plsc.VectorSubcoreMesh(core_axis_name="c", subcore_axis_name="s"),
        scratch_types=[
            pltpu.VMEM((_NCHUNK, _CHUNK), jnp.int32),
            pltpu.VMEM((_CHUNK, _D), jnp.float32),
            pltpu.SemaphoreType.DMA,
        ],
        compiler_params=pltpu.CompilerParams(use_tc_tiling_on_sc=False),
    )
    out = run(embeddings, idx3)
    return out.reshape(inputs.shape[0], inputs.shape[1], _D)


# speedup vs baseline: 1.0467x; 1.0467x over previous
import jax
import jax.numpy as jnp
from jax import lax
from jax.experimental import pallas as pl
from jax.experimental.pallas import tpu as pltpu
from jax.experimental.pallas import tpu_sc as plsc

_NC = 2
_NS = 16
_NW = _NC * _NS

_B = 4096 * 50
_D = 32
_CHUNK = 640
_PER_W = _B // _NW
_NCHUNK = _PER_W // _CHUNK
_NBUF = 5


def _body(table_hbm, idx_hbm, out_hbm, idx_v, rows_v, gsem, wsem):
    wid = lax.axis_index("s") * _NC + lax.axis_index("c")
    base = wid * _PER_W
    pltpu.sync_copy(idx_hbm.at[wid], idx_v)

    def gather(j, b):
        pltpu.async_copy(table_hbm.at[idx_v.at[j]], rows_v.at[b], gsem.at[b])

    for b in range(min(_NBUF, _NCHUNK)):
        gather(b, b)

    def step(j, carry):
        b = lax.rem(j, _NBUF)
        pltpu.make_async_copy(
            table_hbm.at[idx_v.at[j]], rows_v.at[b], gsem.at[b]
        ).wait()
        pltpu.async_copy(
            rows_v.at[b], out_hbm.at[pl.ds(base + j * _CHUNK, _CHUNK)],
            wsem.at[b],
        )

        @pl.when(j + _NBUF < _NCHUNK)
        def _():
            pltpu.make_async_copy(
                rows_v.at[b], out_hbm.at[pl.ds(base + j * _CHUNK, _CHUNK)],
                wsem.at[b],
            ).wait()
            gather(j + _NBUF, b)

        return carry

    lax.fori_loop(0, _NCHUNK, step, 0)

    for j in range(max(_NCHUNK - _NBUF, 0), _NCHUNK):
        b = j % _NBUF
        pltpu.make_async_copy(
            rows_v.at[b], out_hbm.at[pl.ds(base + j * _CHUNK, _CHUNK)],
            wsem.at[b],
        ).wait()


@jax.jit
def kernel(embeddings, inputs):
    idx3 = inputs.reshape(_NW, _NCHUNK, _CHUNK)
    run = pl.kernel(
        _body,
        out_type=jax.ShapeDtypeStruct((_B, _D), jnp.float32),
        mesh=plsc.VectorSubcoreMesh(core_axis_name="c", subcore_axis_name="s"),
        scratch_types=[
            pltpu.VMEM((_NCHUNK, _CHUNK), jnp.int32),
            pltpu.VMEM((_NBUF, _CHUNK, _D), jnp.float32),
            pltpu.SemaphoreType.DMA((_NBUF,)),
            pltpu.SemaphoreType.DMA((_NBUF,)),
        ],
        compiler_params=pltpu.CompilerParams(use_tc_tiling_on_sc=False),
    )
    out = run(embeddings, idx3)
    return out.reshape(inputs.shape[0], inputs.shape[1], _D)


# speedup vs baseline: 1.2283x; 1.1736x over previous
import jax
import jax.numpy as jnp
from jax import lax
from jax.experimental import pallas as pl
from jax.experimental.pallas import tpu as pltpu
from jax.experimental.pallas import tpu_sc as plsc

_NC = 2
_NS = 16
_NW = _NC * _NS

_NB = 4096
_NT = 50
_D = 32
_BLK = _NB // _NW
_PER_W = _BLK * _NT


def _body(table_hbm, idx_hbm, out_hbm, idx_v, sidx_v, ov_v, rows_v, tr_v,
          gsem, wsem):
    wid = lax.axis_index("s") * _NC + lax.axis_index("c")
    b0 = wid * _BLK
    pltpu.sync_copy(idx_hbm.at[pl.ds(b0 * _NT, _PER_W)], idx_v)

    lane = lax.iota(jnp.int32, 16)
    stride_lane = lane * _NT

    def prep(t, p):
        for g in range(_BLK // 16):
            iv = plsc.load_gather(idx_v, [stride_lane + (g * 16 * _NT) + t])
            sidx_v[p, pl.ds(g * 16, 16)] = jax.lax.shift_right_logical(iv, 2)
            ov_v[p, pl.ds(g * 16, 16)] = jax.lax.bitwise_and(iv, 3) * _D

    def fire(t, p):
        pltpu.async_copy(table_hbm.at[sidx_v.at[p]], rows_v.at[p], gsem.at[p])

    def extract(t, p):
        pltpu.make_async_copy(
            table_hbm.at[sidx_v.at[p]], rows_v.at[p], gsem.at[p]
        ).wait()
        @pl.when(t >= 2)
        def _():
            pltpu.make_async_copy(
                tr_v.at[p], out_hbm.at[t, :, pl.ds(b0, _BLK)], wsem.at[p]
            ).wait()
        rv = rows_v.at[p]
        for g in range(_BLK // 16):
            kvec = lane + g * 16
            ov = ov_v[p, pl.ds(g * 16, 16)]
            for c in range(_D):
                vals = plsc.load_gather(rv, [kvec, ov + c])
                tr_v[p, c, pl.ds(g * 16, 16)] = vals
        pltpu.async_copy(
            tr_v.at[p], out_hbm.at[t, :, pl.ds(b0, _BLK)], wsem.at[p]
        )

    prep(0, 0)
    fire(0, 0)

    def step(t, carry):
        p = lax.rem(t, 2)

        @pl.when(t + 1 < _NT)
        def _():
            q = lax.rem(t + 1, 2)
            prep(t + 1, q)
            fire(t + 1, q)

        extract(t, p)
        return carry

    lax.fori_loop(0, _NT, step, 0)

    for t in (_NT - 2, _NT - 1):
        p = t % 2
        pltpu.make_async_copy(
            tr_v.at[p], out_hbm.at[t, :, pl.ds(b0, _BLK)], wsem.at[p]
        ).wait()


@jax.jit
def kernel(embeddings, inputs):
    table2 = embeddings.reshape(250000, 128)
    idxf = inputs.reshape(_NB * _NT)
    run = pl.kernel(
        _body,
        out_type=jax.ShapeDtypeStruct((_NT, _D, _NB), jnp.float32),
        mesh=plsc.VectorSubcoreMesh(core_axis_name="c", subcore_axis_name="s"),
        scratch_types=[
            pltpu.VMEM((_PER_W,), jnp.int32),
            pltpu.VMEM((2, _BLK), jnp.int32),
            pltpu.VMEM((2, _BLK), jnp.int32),
            pltpu.VMEM((2, _BLK, 128), jnp.float32),
            pltpu.VMEM((2, _D, _BLK), jnp.float32),
            pltpu.SemaphoreType.DMA((2,)),
            pltpu.SemaphoreType.DMA((2,)),
        ],
        compiler_params=pltpu.CompilerParams(
            use_tc_tiling_on_sc=True, needs_layout_passes=False
        ),
    )
    out_t = run(table2, idxf)
    return out_t.transpose(2, 0, 1)


# speedup vs baseline: 1.3458x; 1.0956x over previous
import jax
import jax.numpy as jnp
from jax import lax
from jax.experimental import pallas as pl
from jax.experimental.pallas import tpu as pltpu
from jax.experimental.pallas import tpu_sc as plsc

_NC = 2
_NS = 16
_NW = _NC * _NS

_NB = 4096
_NT = 50
_D = 32
_BLK = _NB // _NW
_PER_W = _BLK * _NT


def _body(table_hbm, idx_hbm, out_hbm,
          idx_v, sidx_a, sidx_b, ov_a, ov_b, rows_a, rows_b, tr_a, tr_b,
          gsem, wsem):
    wid = lax.axis_index("s") * _NC + lax.axis_index("c")
    b0 = wid * _BLK
    pltpu.sync_copy(idx_hbm.at[pl.ds(b0 * _NT, _PER_W)], idx_v)

    lane = lax.iota(jnp.int32, 16)
    stride_lane = lane * _NT

    def prep(t, sidx_v, ov_v):
        ivs = [
            plsc.load_gather(idx_v, [stride_lane + (g * 16 * _NT) + t])
            for g in range(_BLK // 16)
        ]
        for g, iv in enumerate(ivs):
            sidx_v[pl.ds(g * 16, 16)] = jax.lax.shift_right_logical(iv, 2)
        for g, iv in enumerate(ivs):
            ov_v[pl.ds(g * 16, 16)] = jax.lax.bitwise_and(iv, 3) * _D

    def fire(sidx_v, rows_v, sem):
        pltpu.async_copy(table_hbm.at[sidx_v], rows_v, sem)

    def extract(t, sidx_v, rows_v, ov_v, tr_v, gs, ws, first):
        pltpu.make_async_copy(table_hbm.at[sidx_v], rows_v, gs).wait()
        if not first:
            pltpu.make_async_copy(
                tr_v, out_hbm.at[t, :, pl.ds(b0, _BLK)], ws
            ).wait()
        for g in range(_BLK // 16):
            kvec = lane + g * 16
            ov = ov_v[pl.ds(g * 16, 16)]
            vals = [
                plsc.load_gather(rows_v, [kvec, ov + c]) for c in range(_D)
            ]
            for c in range(_D):
                tr_v[c, pl.ds(g * 16, 16)] = vals[c]
        pltpu.async_copy(tr_v, out_hbm.at[t, :, pl.ds(b0, _BLK)], ws)

    prep(0, sidx_a, ov_a)
    fire(sidx_a, rows_a, gsem.at[0])

    def step(m, carry):
        t0 = m * 2

        prep(t0 + 1, sidx_b, ov_b)
        fire(sidx_b, rows_b, gsem.at[1])
        extract(t0, sidx_a, rows_a, ov_a, tr_a, gsem.at[0], wsem.at[0], False)

        @pl.when(t0 + 2 < _NT)
        def _():
            prep(t0 + 2, sidx_a, ov_a)
            fire(sidx_a, rows_a, gsem.at[0])

        extract(t0 + 1, sidx_b, rows_b, ov_b, tr_b, gsem.at[1], wsem.at[1], False)
        return carry

    prep(1, sidx_b, ov_b)
    fire(sidx_b, rows_b, gsem.at[1])
    extract(0, sidx_a, rows_a, ov_a, tr_a, gsem.at[0], wsem.at[0], True)
    prep(2, sidx_a, ov_a)
    fire(sidx_a, rows_a, gsem.at[0])
    extract(1, sidx_b, rows_b, ov_b, tr_b, gsem.at[1], wsem.at[1], True)

    lax.fori_loop(1, _NT // 2, step, 0)

    pltpu.make_async_copy(
        tr_a, out_hbm.at[_NT - 2, :, pl.ds(b0, _BLK)], wsem.at[0]
    ).wait()
    pltpu.make_async_copy(
        tr_b, out_hbm.at[_NT - 1, :, pl.ds(b0, _BLK)], wsem.at[1]
    ).wait()


@jax.jit
def kernel(embeddings, inputs):
    table2 = embeddings.reshape(250000, 128)
    idxf = inputs.reshape(_NB * _NT)
    run = pl.kernel(
        _body,
        out_type=jax.ShapeDtypeStruct((_NT, _D, _NB), jnp.float32),
        mesh=plsc.VectorSubcoreMesh(core_axis_name="c", subcore_axis_name="s"),
        scratch_types=[
            pltpu.VMEM((_PER_W,), jnp.int32),
            pltpu.VMEM((_BLK,), jnp.int32),
            pltpu.VMEM((_BLK,), jnp.int32),
            pltpu.VMEM((_BLK,), jnp.int32),
            pltpu.VMEM((_BLK,), jnp.int32),
            pltpu.VMEM((_BLK, 128), jnp.float32),
            pltpu.VMEM((_BLK, 128), jnp.float32),
            pltpu.VMEM((_D, _BLK), jnp.float32),
            pltpu.VMEM((_D, _BLK), jnp.float32),
            pltpu.SemaphoreType.DMA((2,)),
            pltpu.SemaphoreType.DMA((2,)),
        ],
        compiler_params=pltpu.CompilerParams(
            use_tc_tiling_on_sc=True, needs_layout_passes=False
        ),
    )
    out_t = run(table2, idxf)
    return out_t.transpose(2, 0, 1)
